# baseline (speedup 1.0000x reference)
import functools
import jax
import jax.numpy as jnp
from jax import lax
from jax.experimental import pallas as pl
from jax.experimental.pallas import tpu as pltpu
from jax.experimental.pallas import tpu_sc as plsc

_N_POINTS = 100000
_B, _H, _W, _K = 4, 384, 384, 8
_M = _B * _H * _W
_HWK = _H * _W * _K
_N_CH = 4
_G = 8
_PPT = _M // _G
_RPB = 2
_PBLK = _RPB * _W
_NBLK = _PPT // _PBLK
_NGRP = _W // 16


def _build_sc_kernel():
    mesh = plsc.VectorSubcoreMesh(core_axis_name="c", subcore_axis_name="s")

    @functools.partial(
        pl.kernel,
        out_type=jax.ShapeDtypeStruct((_B, _H, 3, _N_CH, 128), jnp.float32),
        mesh=mesh,
        scratch_types=[
            pltpu.VMEM((_N_POINTS,), jnp.float32),
            pltpu.VMEM((_RPB, 3, _K, 128), jnp.float32),
            pltpu.VMEM((_RPB, 3, _K, 128), jnp.float32),
            pltpu.VMEM((_RPB, 3, _K, 128), jnp.float32),
            pltpu.VMEM((_RPB, 3, _K, 128), jnp.float32),
            pltpu.VMEM((_RPB, 3, 128), jnp.float32),
            pltpu.VMEM((_RPB, 3, 128), jnp.float32),
            pltpu.VMEM((16,), jnp.float32),
            pltpu.SemaphoreType.DMA,
            pltpu.SemaphoreType.DMA,
            pltpu.SemaphoreType.DMA,
            pltpu.SemaphoreType.DMA,
            pltpu.SemaphoreType.DMA,
            pltpu.SemaphoreType.DMA,
        ],
        compiler_params=pltpu.CompilerParams(
            needs_layout_passes=False, use_tc_tiling_on_sc=False),
    )
    def gauss_sc(table_flat, dii_t, c16_hbm, out_hbm, col, dbuf0, dbuf1,
                 ibuf0, ibuf1, obuf0, obuf1, cvm, sd0, sd1, si0, si1,
                 so0, so1):
        wid = lax.axis_index("s") * 2 + lax.axis_index("c")
        ch = wid % _N_CH
        g = wid // _N_CH
        b = g // 2
        h0 = (g % 2) * (_PPT // _W)

        dbufs, ibufs, obufs = (dbuf0, dbuf1), (ibuf0, ibuf1), (obuf0, obuf1)
        sds, sis, sos = (sd0, sd1), (si0, si1), (so0, so1)

        def d_src(blk):
            return dii_t.at[b, 0, pl.ds(h0 + blk * _RPB, _RPB), :, :, :]

        def i_src(blk):
            return dii_t.at[b, 1, pl.ds(h0 + blk * _RPB, _RPB), :, :, :]

        def o_dst(blk):
            return out_hbm.at[b, pl.ds(h0 + blk * _RPB, _RPB), :, ch, :]

        pltpu.sync_copy(table_flat.at[pl.ds(ch * _N_POINTS, _N_POINTS)], col)
        pltpu.sync_copy(c16_hbm, cvm)
        cv = cvm[...]
        scale = -0.5 / (cv * cv)
        zero = jnp.zeros((16,), jnp.float32)

        def compute(dbuf, ibuf, obuf):
            for r in range(_RPB):
                @plsc.parallel_loop(0, _W, 16, unroll=3)
                def _grp(w0, r=r):
                    wt = lax.shift_right_logical(w0, 7)
                    wl = lax.bitwise_and(w0, 127)
                    ds = zero
                    acc = zero
                    for k in range(_K):
                        dk = dbuf[r, wt, k, pl.ds(wl, 16)]
                        w = jnp.exp(dk * dk * scale)
                        ds = ds + w
                        ik = ibuf[r, wt, k, pl.ds(wl, 16)].astype(jnp.int32)
                        xk = plsc.load_gather(col, [ik])
                        acc = acc + w * xk
                    res = acc / (ds + 0.001)
                    obuf[r, wt, pl.ds(wl, 16)] = jnp.where(ds > 0, res, 0.0)

        for s in range(2):
            pltpu.async_copy(d_src(s), dbufs[s], sds[s])
            pltpu.async_copy(i_src(s), ibufs[s], sis[s])

        def outer(i, carry):
            for s in range(2):
                blk = i * 2 + s
                pltpu.make_async_copy(d_src(blk), dbufs[s], sds[s]).wait()
                pltpu.make_async_copy(i_src(blk), ibufs[s], sis[s]).wait()

                @pl.when(blk >= 2)
                def _():
                    pltpu.make_async_copy(obufs[s], o_dst(blk - 2),
                                          sos[s]).wait()

                compute(dbufs[s], ibufs[s], obufs[s])
                pltpu.async_copy(obufs[s], o_dst(blk), sos[s])

                @pl.when(blk + 2 < _NBLK)
                def _():
                    pltpu.async_copy(d_src(blk + 2), dbufs[s], sds[s])
                    pltpu.async_copy(i_src(blk + 2), ibufs[s], sis[s])
            return carry

        lax.fori_loop(0, _NBLK // 2, outer, 0)
        pltpu.make_async_copy(obuf0, o_dst(_NBLK - 2), so0).wait()
        pltpu.make_async_copy(obuf1, o_dst(_NBLK - 1), so1).wait()

    return gauss_sc


_GAUSS_SC = _build_sc_kernel()


@jax.jit
def kernel(spatial_rgb, dist_and_index_list, c):
    table_flat = spatial_rgb.T.reshape(-1)
    dii_t = jnp.transpose(
        dist_and_index_list.reshape(_B, 2, _H, 3, 128, _K),
        (0, 1, 2, 3, 5, 4))
    c16 = jnp.broadcast_to(c.reshape(1), (16,)).astype(jnp.float32)
    out6 = _GAUSS_SC(table_flat, dii_t, c16)
    return jnp.transpose(out6, (0, 1, 2, 4, 3)).reshape(_B, _H, _W, _N_CH)


# speedup vs baseline: 1.3976x; 1.3976x over previous
import functools
import jax
import jax.numpy as jnp
from jax import lax
from jax.experimental import pallas as pl
from jax.experimental.pallas import tpu as pltpu
from jax.experimental.pallas import tpu_sc as plsc

_N_POINTS = 100000
_B, _H, _W, _K = 4, 384, 384, 8
_M = _B * _H * _W
_HWK = _H * _W * _K
_N_CH = 4
_G = 8
_PPT = _M // _G
_RPB = 2
_PBLK = _RPB * _W
_NBLK = _PPT // _PBLK
_NGRP = _W // 16


def _build_sc_kernel():
    mesh = plsc.VectorSubcoreMesh(core_axis_name="c", subcore_axis_name="s")

    @functools.partial(
        pl.kernel,
        out_type=jax.ShapeDtypeStruct((_B, _H, 3, _N_CH, 128), jnp.float32),
        mesh=mesh,
        scratch_types=[
            pltpu.VMEM((_N_POINTS,), jnp.float32),
            pltpu.VMEM((_RPB, 3, _K, 128), jnp.float32),
            pltpu.VMEM((_RPB, 3, _K, 128), jnp.float32),
            pltpu.VMEM((_RPB, 3, _K, 128), jnp.float32),
            pltpu.VMEM((_RPB, 3, _K, 128), jnp.float32),
            pltpu.VMEM((_RPB, 3, 128), jnp.float32),
            pltpu.VMEM((_RPB, 3, 128), jnp.float32),
            pltpu.VMEM((16,), jnp.float32),
            pltpu.SemaphoreType.DMA,
            pltpu.SemaphoreType.DMA,
            pltpu.SemaphoreType.DMA,
            pltpu.SemaphoreType.DMA,
            pltpu.SemaphoreType.DMA,
            pltpu.SemaphoreType.DMA,
        ],
        compiler_params=pltpu.CompilerParams(
            needs_layout_passes=False, use_tc_tiling_on_sc=False),
    )
    def gauss_sc(table_flat, dii_t, c16_hbm, out_hbm, col, dbuf0, dbuf1,
                 ibuf0, ibuf1, obuf0, obuf1, cvm, sd0, sd1, si0, si1,
                 so0, so1):
        wid = lax.axis_index("s") * 2 + lax.axis_index("c")
        ch = wid % _N_CH
        g = wid // _N_CH
        b = g // 2
        h0 = (g % 2) * (_PPT // _W)

        dbufs, ibufs, obufs = (dbuf0, dbuf1), (ibuf0, ibuf1), (obuf0, obuf1)
        sds, sis, sos = (sd0, sd1), (si0, si1), (so0, so1)

        def d_src(blk):
            return dii_t.at[b, 0, pl.ds(h0 + blk * _RPB, _RPB), :, :, :]

        def i_src(blk):
            return dii_t.at[b, 1, pl.ds(h0 + blk * _RPB, _RPB), :, :, :]

        def o_dst(blk):
            return out_hbm.at[b, pl.ds(h0 + blk * _RPB, _RPB), :, ch, :]

        pltpu.sync_copy(table_flat.at[pl.ds(ch * _N_POINTS, _N_POINTS)], col)
        pltpu.sync_copy(c16_hbm, cvm)
        cv = cvm[...]
        scale = -0.5 / (cv * cv)
        zero = jnp.zeros((16,), jnp.float32)

        def compute(dbuf, ibuf, obuf):
            if True:
                @plsc.parallel_loop(0, _PBLK, 16, unroll=2)
                def _grp(p0):
                    r = jnp.where(p0 >= _W, 1, 0)
                    w0 = p0 - r * _W
                    wt = lax.shift_right_logical(w0, 7)
                    wl = lax.bitwise_and(w0, 127)
                    ds = zero
                    acc = zero
                    for k in range(_K):
                        dk = dbuf[r, wt, k, pl.ds(wl, 16)]
                        w = jnp.exp(dk * dk * scale)
                        ds = ds + w
                        ik = ibuf[r, wt, k, pl.ds(wl, 16)].astype(jnp.int32)
                        xk = plsc.load_gather(col, [ik])
                        acc = acc + w * xk
                    res = acc / (ds + 0.001)
                    obuf[r, wt, pl.ds(wl, 16)] = jnp.where(ds > 0, res, 0.0)

        for s in range(2):
            pltpu.async_copy(d_src(s), dbufs[s], sds[s])
            pltpu.async_copy(i_src(s), ibufs[s], sis[s])

        def outer(i, carry):
            for s in range(2):
                blk = i * 2 + s
                pltpu.make_async_copy(d_src(blk), dbufs[s], sds[s]).wait()
                pltpu.make_async_copy(i_src(blk), ibufs[s], sis[s]).wait()

                @pl.when(blk >= 2)
                def _():
                    pltpu.make_async_copy(obufs[s], o_dst(blk - 2),
                                          sos[s]).wait()

                compute(dbufs[s], ibufs[s], obufs[s])
                pltpu.async_copy(obufs[s], o_dst(blk), sos[s])

                @pl.when(blk + 2 < _NBLK)
                def _():
                    pltpu.async_copy(d_src(blk + 2), dbufs[s], sds[s])
                    pltpu.async_copy(i_src(blk + 2), ibufs[s], sis[s])
            return carry

        lax.fori_loop(0, _NBLK // 2, outer, 0)
        pltpu.make_async_copy(obuf0, o_dst(_NBLK - 2), so0).wait()
        pltpu.make_async_copy(obuf1, o_dst(_NBLK - 1), so1).wait()

    return gauss_sc


_GAUSS_SC = _build_sc_kernel()


@jax.jit
def kernel(spatial_rgb, dist_and_index_list, c):
    table_flat = spatial_rgb.T.reshape(-1)
    dii_t = jnp.transpose(
        dist_and_index_list.reshape(_B, 2, _H, 3, 128, _K),
        (0, 1, 2, 3, 5, 4))
    c16 = jnp.broadcast_to(c.reshape(1), (16,)).astype(jnp.float32)
    out6 = _GAUSS_SC(table_flat, dii_t, c16)
    return jnp.transpose(out6, (0, 1, 2, 4, 3)).reshape(_B, _H, _W, _N_CH)


# speedup vs baseline: 1.3979x; 1.0002x over previous
import functools
import jax
import jax.numpy as jnp
from jax import lax
from jax.experimental import pallas as pl
from jax.experimental.pallas import tpu as pltpu
from jax.experimental.pallas import tpu_sc as plsc

_N_POINTS = 100000
_B, _H, _W, _K = 4, 384, 384, 8
_M = _B * _H * _W
_N_CH = 4
_G = 8
_PPT = _M // _G
_RPB = 2
_PBLK = _RPB * _W
_NBLK = _PPT // _PBLK


def _build_sc_kernel():
    mesh = plsc.VectorSubcoreMesh(core_axis_name="c", subcore_axis_name="s")

    @functools.partial(
        pl.kernel,
        out_type=jax.ShapeDtypeStruct((_B, _H, 3, _N_CH, 128), jnp.float32),
        mesh=mesh,
        scratch_types=[
            pltpu.VMEM((_N_POINTS,), jnp.float32),
            pltpu.VMEM((_RPB, 3, _K, 128), jnp.float32),
            pltpu.VMEM((_RPB, 3, _K, 128), jnp.float32),
            pltpu.VMEM((_RPB, 3, _K, 128), jnp.float32),
            pltpu.VMEM((_RPB, 3, _K, 128), jnp.float32),
            pltpu.VMEM((_RPB, 3, 128), jnp.float32),
            pltpu.VMEM((_RPB, 3, 128), jnp.float32),
            pltpu.VMEM((16,), jnp.float32),
            pltpu.SemaphoreType.DMA,
            pltpu.SemaphoreType.DMA,
            pltpu.SemaphoreType.DMA,
            pltpu.SemaphoreType.DMA,
            pltpu.SemaphoreType.DMA,
            pltpu.SemaphoreType.DMA,
        ],
        compiler_params=pltpu.CompilerParams(
            needs_layout_passes=False, use_tc_tiling_on_sc=False),
    )
    def gauss_sc(table_flat, dii_t, c16_hbm, out_hbm, col, dbuf0, dbuf1,
                 ibuf0, ibuf1, obuf0, obuf1, cvm, sd0, sd1, si0, si1,
                 so0, so1):
        wid = lax.axis_index("s") * 2 + lax.axis_index("c")
        ch = wid % _N_CH
        g = wid // _N_CH
        b = g // 2
        h0 = (g % 2) * (_PPT // _W)

        dbufs, ibufs, obufs = (dbuf0, dbuf1), (ibuf0, ibuf1), (obuf0, obuf1)
        sds, sis, sos = (sd0, sd1), (si0, si1), (so0, so1)

        def d_src(blk):
            return dii_t.at[b, 0, pl.ds(h0 + blk * _RPB, _RPB), :, :, :]

        def i_src(blk):
            return dii_t.at[b, 1, pl.ds(h0 + blk * _RPB, _RPB), :, :, :]

        def o_dst(blk):
            return out_hbm.at[b, pl.ds(h0 + blk * _RPB, _RPB), :, ch, :]

        pltpu.sync_copy(table_flat.at[pl.ds(ch * _N_POINTS, _N_POINTS)], col)
        pltpu.sync_copy(c16_hbm, cvm)
        cv = cvm[...]
        scale = -0.5 / (cv * cv)
        zero = jnp.zeros((16,), jnp.float32)

        def compute(dbuf, ibuf, obuf):
            @plsc.parallel_loop(0, _PBLK, 16, unroll=2)
            def _grp(p0):
                r = jnp.where(p0 >= _W, 1, 0)
                w0 = p0 - r * _W
                wt = lax.shift_right_logical(w0, 7)
                wl = lax.bitwise_and(w0, 127)
                ds = zero
                acc = zero
                for k in range(_K):
                    dk = dbuf[r, wt, k, pl.ds(wl, 16)]
                    w = jnp.exp(dk * dk * scale)
                    ds = ds + w
                    ik = ibuf[r, wt, k, pl.ds(wl, 16)].astype(jnp.int32)
                    xk = plsc.load_gather(col, [ik])
                    acc = acc + w * xk
                res = acc / (ds + 0.001)
                obuf[r, wt, pl.ds(wl, 16)] = jnp.where(ds > 0, res, 0.0)

        for s in range(2):
            pltpu.async_copy(d_src(s), dbufs[s], sds[s])
            pltpu.async_copy(i_src(s), ibufs[s], sis[s])

        def outer(i, carry):
            for s in range(2):
                blk = i * 2 + s
                pltpu.make_async_copy(d_src(blk), dbufs[s], sds[s]).wait()
                pltpu.make_async_copy(i_src(blk), ibufs[s], sis[s]).wait()

                @pl.when(blk >= 2)
                def _():
                    pltpu.make_async_copy(obufs[s], o_dst(blk - 2),
                                          sos[s]).wait()

                compute(dbufs[s], ibufs[s], obufs[s])
                pltpu.async_copy(obufs[s], o_dst(blk), sos[s])

                @pl.when(blk + 2 < _NBLK)
                def _():
                    pltpu.async_copy(d_src(blk + 2), dbufs[s], sds[s])
                    pltpu.async_copy(i_src(blk + 2), ibufs[s], sis[s])
            return carry

        lax.fori_loop(0, _NBLK // 2, outer, 0)
        pltpu.make_async_copy(obuf0, o_dst(_NBLK - 2), so0).wait()
        pltpu.make_async_copy(obuf1, o_dst(_NBLK - 1), so1).wait()

    return gauss_sc


_GAUSS_SC = _build_sc_kernel()


@jax.jit
def kernel(spatial_rgb, dist_and_index_list, c):
    table_flat = spatial_rgb.T.reshape(-1)
    dii_t = jnp.transpose(
        dist_and_index_list.reshape(_B, 2, _H, 3, 128, _K),
        (0, 1, 2, 3, 5, 4))
    c16 = jnp.broadcast_to(c.reshape(1), (16,)).astype(jnp.float32)
    out6 = _GAUSS_SC(table_flat, dii_t, c16)
    return jnp.transpose(out6, (0, 1, 2, 4, 3)).reshape(_B, _H, _W, _N_CH)
